# baseline (speedup 1.0000x reference)
import functools

import jax
import jax.numpy as jnp
from jax import lax
from jax.experimental import pallas as pl
from jax.experimental.pallas import tpu as pltpu
from jax.experimental.pallas import tpu_sc as plsc

_L = 128
_RING = 2
_BLK = 80


@functools.lru_cache(maxsize=None)
def _build_gather(B, D):
    info = plsc.get_sparse_core_info()
    NC, NS = info.num_cores, info.num_subcores
    NW = NC * NS
    FR = B // _L
    assert B % _L == 0 and FR % NW == 0
    fr_per_w = FR // NW
    assert fr_per_w % _BLK == 0 and _BLK % _RING == 0
    n_blocks = fr_per_w // _BLK
    n_groups = _BLK // _RING

    mesh = plsc.VectorSubcoreMesh(core_axis_name="c", subcore_axis_name="s")

    @functools.partial(
        pl.kernel,
        mesh=mesh,
        out_type=jax.ShapeDtypeStruct((B, D), jnp.float32),
        scratch_types=[
            pltpu.VMEM((_BLK, _L), jnp.int32),
            [pltpu.VMEM((_L, 2 * D), jnp.float32) for _ in range(_RING)],
            [pltpu.VMEM((_L, D), jnp.float32) for _ in range(_RING)],
            [pltpu.SemaphoreType.DMA for _ in range(_RING)],
            [pltpu.SemaphoreType.DMA for _ in range(_RING)],
        ],
        compiler_params=pltpu.CompilerParams(use_tc_tiling_on_sc=True),
    )
    def gather_kernel(idx_hbm, table_hbm, out_hbm, idx_v, rows, cmp,
                      sg, sw):
        wid = lax.axis_index("s") * NC + lax.axis_index("c")
        base_fr = wid * fr_per_w

        def block(bi, carry):
            fr0 = base_fr + bi * _BLK
            pltpu.sync_copy(idx_hbm.at[pl.ds(fr0, _BLK)], idx_v)

            def group(q, carry):
                r0 = q * _RING

                @pl.when(q > 0)
                def _():
                    for j in range(_RING):
                        pltpu.make_async_copy(
                            cmp[j], out_hbm.at[pl.ds((fr0 + r0 - _RING + j)
                                                     * _L, _L)], sw[j]).wait()

                for j in range(_RING):
                    pltpu.async_copy(table_hbm.at[idx_v.at[r0 + j]], rows[j],
                                     sg[j])
                for j in range(_RING):
                    fr = fr0 + r0 + j
                    pltpu.make_async_copy(table_hbm.at[idx_v.at[r0 + j]],
                                          rows[j], sg[j]).wait()

                    def compact(i, c, rj=rows[j], cj=cmp[j]):
                        r4 = i * 8
                        for rr in range(8):
                            for cc in range(D // 16):
                                cj[r4 + rr, pl.ds(cc * 16, 16)] = \
                                    rj[r4 + rr, pl.ds(cc * 16, 16)]
                        return c

                    lax.fori_loop(0, _L // 8, compact, 0)
                    pltpu.async_copy(cmp[j], out_hbm.at[pl.ds(fr * _L, _L)],
                                     sw[j])
                return carry

            lax.fori_loop(0, n_groups, group, 0)
            for j in range(_RING):
                fr = fr0 + _BLK - _RING + j
                pltpu.make_async_copy(cmp[j],
                                      out_hbm.at[pl.ds(fr * _L, _L)],
                                      sw[j]).wait()
            return carry

        lax.fori_loop(0, n_blocks, block, 0)

    return gather_kernel


def kernel(x, table):
    bsz, hist = x.shape
    d = table.shape[1]
    b = bsz * hist
    idx = x.reshape(b // _L, _L).astype(jnp.int32)
    tab2 = jnp.concatenate([table, table], axis=1)
    out = _build_gather(b, d)(idx, tab2)
    return out.reshape(bsz, hist, d)


# speedup vs baseline: 1.2527x; 1.2527x over previous
import functools

import jax
import jax.numpy as jnp
from jax import lax
from jax.experimental import pallas as pl
from jax.experimental.pallas import tpu as pltpu
from jax.experimental.pallas import tpu_sc as plsc

_L = 128
_RING = 2
_BLK = 80


@functools.lru_cache(maxsize=None)
def _build_gather(B, D):
    info = plsc.get_sparse_core_info()
    NC, NS = info.num_cores, info.num_subcores
    NW = NC * NS
    FR = B // _L
    assert B % _L == 0 and FR % NW == 0
    fr_per_w = FR // NW
    assert fr_per_w % _BLK == 0 and _BLK % _RING == 0
    n_blocks = fr_per_w // _BLK
    n_groups = _BLK // _RING

    mesh = plsc.VectorSubcoreMesh(core_axis_name="c", subcore_axis_name="s")

    @functools.partial(
        pl.kernel,
        mesh=mesh,
        out_type=jax.ShapeDtypeStruct((B, D), jnp.float32),
        scratch_types=[
            pltpu.VMEM((_BLK, _L), jnp.int32),
            [pltpu.VMEM((_L, 2 * D), jnp.float32) for _ in range(_RING)],
            [pltpu.VMEM((_L, D), jnp.float32) for _ in range(_RING)],
            pltpu.VMEM_SHARED((2048, 2 * D), jnp.float32),
            [pltpu.SemaphoreType.DMA for _ in range(_RING)],
            [pltpu.SemaphoreType.DMA for _ in range(_RING)],
        ],
        compiler_params=pltpu.CompilerParams(use_tc_tiling_on_sc=True),
    )
    def gather_kernel(idx_hbm, table_hbm, out_hbm, idx_v, rows, cmp,
                      shared_tab, sg, sw):
        wid = lax.axis_index("s") * NC + lax.axis_index("c")
        base_fr = wid * fr_per_w

        @pl.when(lax.axis_index("s") == 0)
        def _():
            pltpu.sync_copy(table_hbm, shared_tab)

        plsc.subcore_barrier()

        def block(bi, carry):
            fr0 = base_fr + bi * _BLK
            pltpu.sync_copy(idx_hbm.at[pl.ds(fr0, _BLK)], idx_v)

            def group(q, carry):
                r0 = q * _RING

                @pl.when(q > 0)
                def _():
                    for j in range(_RING):
                        pltpu.make_async_copy(
                            cmp[j], out_hbm.at[pl.ds((fr0 + r0 - _RING + j)
                                                     * _L, _L)], sw[j]).wait()

                for j in range(_RING):
                    pltpu.async_copy(shared_tab.at[idx_v.at[r0 + j]], rows[j],
                                     sg[j])
                for j in range(_RING):
                    fr = fr0 + r0 + j
                    pltpu.make_async_copy(shared_tab.at[idx_v.at[r0 + j]],
                                          rows[j], sg[j]).wait()

                    def compact(i, c, rj=rows[j], cj=cmp[j]):
                        r4 = i * 8
                        for rr in range(8):
                            for cc in range(D // 16):
                                cj[r4 + rr, pl.ds(cc * 16, 16)] = \
                                    rj[r4 + rr, pl.ds(cc * 16, 16)]
                        return c

                    lax.fori_loop(0, _L // 8, compact, 0)
                    pltpu.async_copy(cmp[j], out_hbm.at[pl.ds(fr * _L, _L)],
                                     sw[j])
                return carry

            lax.fori_loop(0, n_groups, group, 0)
            for j in range(_RING):
                fr = fr0 + _BLK - _RING + j
                pltpu.make_async_copy(cmp[j],
                                      out_hbm.at[pl.ds(fr * _L, _L)],
                                      sw[j]).wait()
            return carry

        lax.fori_loop(0, n_blocks, block, 0)

    return gather_kernel


def kernel(x, table):
    bsz, hist = x.shape
    d = table.shape[1]
    b = bsz * hist
    idx = x.reshape(b // _L, _L).astype(jnp.int32)
    tab2 = jnp.concatenate([table, table], axis=1)
    out = _build_gather(b, d)(idx, tab2)
    return out.reshape(bsz, hist, d)
